# Initial kernel scaffold; baseline (speedup 1.0000x reference)
import jax
import jax.numpy as jnp
from jax.experimental import pallas as pl


def kernel(x, edge_index, W1_0, W1_1, W1_2, b1, W2_0, W2_1, W2_2, b2, Wlin, blin):
    raise NotImplementedError("write your pallas kernel here")


# speedup vs baseline: 3.2832x; 3.2832x over previous
import functools

import jax
import jax.numpy as jnp
from jax import lax
from jax.experimental import pallas as pl
from jax.experimental.pallas import tpu as pltpu
from jax.experimental.pallas import tpu_sc as plsc

N = 10000
E = 160000
D_IN = 256
D_HID = 512
D_OUT = 40

NP = 10240
L = 16
NC = 2
NS = 16
EPT = E // NS
EPW = E // (NC * NS)
CH = 80
SUP = 400
ROWS_PER_TILE = NP // NS

_mesh = plsc.VectorSubcoreMesh(core_axis_name="c", subcore_axis_name="s")


def _i32v(val):
    return lax.broadcast(jnp.int32(val), (L,))


def _f32v(val):
    return lax.broadcast(jnp.float32(val), (L,))


def _rsqrt16(d):
    xi = plsc.bitcast(d, jnp.int32)
    mi = _i32v(0x5F3759DF) - lax.shift_right_arithmetic(xi, _i32v(1))
    y = plsc.bitcast(mi, jnp.float32)
    for _ in range(3):
        y = y * (_f32v(1.5) - _f32v(0.5) * d * y * y)
    return y


def _norm_body(row_hbm, col_hbm, norm_hbm, norm2_hbm,
               deg_sh, zv, ones_v, idx_v, row_v, col_v, dinv_v, nrm_v, nrm2_v):
    c = lax.axis_index("c")
    s = lax.axis_index("s")

    zero16 = _f32v(0.0)
    one16 = _f32v(1.0)

    def fill_z(i, carry):
        zv[pl.ds(i * L, L)] = zero16
        return carry
    lax.fori_loop(0, ROWS_PER_TILE // L, fill_z, 0)
    for i in range(CH // L):
        ones_v[pl.ds(i * L, L)] = one16

    pltpu.sync_copy(zv, deg_sh.at[pl.ds(s * ROWS_PER_TILE, ROWS_PER_TILE)])
    plsc.subcore_barrier()

    def deg_chunk(k, carry):
        off = s * EPT + k * CH
        pltpu.sync_copy(row_hbm.at[pl.ds(off, CH)], idx_v)
        pltpu.sync_copy(ones_v, deg_sh.at[idx_v], add=True)
        return carry
    lax.fori_loop(0, EPT // CH, deg_chunk, 0)
    plsc.subcore_barrier()

    pltpu.sync_copy(deg_sh, dinv_v)

    def dloop(i, carry):
        d = dinv_v[pl.ds(i * L, L)]
        y = _rsqrt16(d)
        dinv_v[pl.ds(i * L, L)] = jnp.where(d > _f32v(0.5), y, zero16)
        return carry
    lax.fori_loop(0, NP // L, dloop, 0)

    w = s * NC + c
    base = w * EPW
    lane = lax.iota(jnp.int32, L)
    tail_mask = lane < _i32v(8)

    def compute_norm(r16, c16, mask):
        dr = plsc.load_gather(dinv_v, [r16], mask=mask)
        dc = plsc.load_gather(dinv_v, [c16], mask=mask)
        nv = -(dr * dc)
        return jnp.where(r16 == c16, nv - one16, nv)

    def nchunk(k, carry):
        off = base + k * 1000
        pltpu.sync_copy(row_hbm.at[pl.ds(off, 1000)], row_v.at[pl.ds(0, 1000)])
        pltpu.sync_copy(col_hbm.at[pl.ds(off, 1000)], col_v.at[pl.ds(0, 1000)])

        def inner(j, cc):
            r16 = row_v[pl.ds(j * L, L)]
            c16 = col_v[pl.ds(j * L, L)]
            nv = compute_norm(r16, c16, None)
            nrm_v[pl.ds(j * L, L)] = nv
            nrm2_v[pl.ds(j * L, L)] = nv + nv
            return cc
        lax.fori_loop(0, 62, inner, 0)
        r16 = row_v[pl.ds(992, L)]
        c16 = col_v[pl.ds(992, L)]
        nv = compute_norm(r16, c16, tail_mask)
        nrm_v[pl.ds(992, L)] = nv
        nrm2_v[pl.ds(992, L)] = nv + nv

        pltpu.sync_copy(nrm_v.at[pl.ds(0, 1000)], norm_hbm.at[pl.ds(off, 1000)])
        pltpu.sync_copy(nrm2_v.at[pl.ds(0, 1000)], norm2_hbm.at[pl.ds(off, 1000)])
        return carry
    lax.fori_loop(0, EPW // 1000, nchunk, 0)


_norm_kernel = pl.kernel(
    _norm_body,
    out_type=(jax.ShapeDtypeStruct((E,), jnp.float32),
              jax.ShapeDtypeStruct((E,), jnp.float32)),
    mesh=_mesh,
    compiler_params=pltpu.CompilerParams(needs_layout_passes=False),
    scratch_types=[
        pltpu.VMEM_SHARED((NP,), jnp.float32),
        pltpu.VMEM((ROWS_PER_TILE,), jnp.float32),
        pltpu.VMEM((CH,), jnp.float32),
        pltpu.VMEM((CH,), jnp.int32),
        pltpu.VMEM((1008,), jnp.int32),
        pltpu.VMEM((1008,), jnp.int32),
        pltpu.VMEM((NP,), jnp.float32),
        pltpu.VMEM((1008,), jnp.float32),
        pltpu.VMEM((1008,), jnp.float32),
    ],
)


def _prop_body(nb, h_hbm, row_hbm, col_hbm, nrm_hbm, out_hbm,
               acc_sh, zrows_v, row_v, nrm_v, col_v, gidx_v, rows_v):
    c = lax.axis_index("c")
    s = lax.axis_index("s")
    zero16 = _f32v(0.0)

    def zb(i, carry):
        for g in range(128 // L):
            zrows_v[i, pl.ds(g * L, L)] = zero16
        return carry
    lax.fori_loop(0, 64, zb, 0)

    for bi in range(nb // NC):
        b = c + NC * bi
        for t in range(ROWS_PER_TILE // 64):
            pltpu.sync_copy(
                zrows_v, acc_sh.at[pl.ds(s * ROWS_PER_TILE + t * 64, 64), :])
        plsc.subcore_barrier()

        bvec = lax.broadcast(b * NP, (L,))
        base = s * EPT

        def schunk(k, carry):
            off = base + k * SUP
            pltpu.sync_copy(row_hbm.at[pl.ds(off, SUP)], row_v)
            pltpu.sync_copy(nrm_hbm.at[pl.ds(off, SUP)], nrm_v)
            for t in range(SUP // CH):
                pltpu.sync_copy(col_hbm.at[pl.ds(off + t * CH, CH)], col_v)

                def gix(j, cc):
                    r16 = row_v[pl.ds(t * CH + j * L, L)]
                    gidx_v[pl.ds(j * L, L)] = r16 + bvec
                    return cc
                lax.fori_loop(0, CH // L, gix, 0)

                pltpu.sync_copy(h_hbm.at[gidx_v], rows_v)

                def scale(e, cc):
                    nbv = plsc.load_gather(
                        nrm_v, [lax.broadcast(t * CH + e, (L,))])
                    for g in range(128 // L):
                        v = rows_v[e, pl.ds(g * L, L)]
                        rows_v[e, pl.ds(g * L, L)] = v * nbv
                    return cc
                lax.fori_loop(0, CH, scale, 0)

                pltpu.sync_copy(rows_v, acc_sh.at[col_v], add=True)
            return carry
        lax.fori_loop(0, EPT // SUP, schunk, 0)
        plsc.subcore_barrier()

        pltpu.sync_copy(
            acc_sh.at[pl.ds(s * ROWS_PER_TILE, ROWS_PER_TILE), :],
            out_hbm.at[pl.ds(b * NP + s * ROWS_PER_TILE, ROWS_PER_TILE), :])


def _make_prop(nb):
    return pl.kernel(
        functools.partial(_prop_body, nb),
        out_type=jax.ShapeDtypeStruct((nb * NP, 128), jnp.float32),
        mesh=_mesh,
        compiler_params=pltpu.CompilerParams(needs_layout_passes=False),
        scratch_types=[
            pltpu.VMEM_SHARED((NP, 128), jnp.float32),
            pltpu.VMEM((64, 128), jnp.float32),
            pltpu.VMEM((SUP,), jnp.int32),
            pltpu.VMEM((SUP,), jnp.float32),
            pltpu.VMEM((CH,), jnp.int32),
            pltpu.VMEM((CH,), jnp.int32),
            pltpu.VMEM((CH, 128), jnp.float32),
        ],
    )


_prop2 = _make_prop(2)
_prop4 = _make_prop(4)


BM = 512


def _mm3_body(x0_ref, x1_ref, x2_ref, wa_ref, wb_ref, wc_ref, b_ref, o_ref):
    nbin = x0_ref.shape[0]
    x0 = jnp.concatenate([x0_ref[k] for k in range(nbin)], axis=1)
    x1 = jnp.concatenate([x1_ref[k] for k in range(nbin)], axis=1)
    x2 = jnp.concatenate([x2_ref[k] for k in range(nbin)], axis=1)
    wc = wc_ref[...]
    acc = jnp.dot(x0, wa_ref[...] - wc, preferred_element_type=jnp.float32)
    acc = acc + jnp.dot(x1, wb_ref[...], preferred_element_type=jnp.float32)
    acc = acc + jnp.dot(x2, wc, preferred_element_type=jnp.float32)
    acc = acc + b_ref[...]
    h = jnp.maximum(acc, 0.0)
    for k in range(o_ref.shape[0]):
        o_ref[k] = h[:, k * 128:(k + 1) * 128]


def _mm3(x0_blk, x1_blk, x2_blk, wa, wb, wc, bias):
    nbin = x0_blk.shape[0]
    din = nbin * 128
    return pl.pallas_call(
        _mm3_body,
        grid=(NP // BM,),
        in_specs=[
            pl.BlockSpec((nbin, BM, 128), lambda i: (0, i, 0)),
            pl.BlockSpec((nbin, BM, 128), lambda i: (0, i, 0)),
            pl.BlockSpec((nbin, BM, 128), lambda i: (0, i, 0)),
            pl.BlockSpec((din, D_HID), lambda i: (0, 0)),
            pl.BlockSpec((din, D_HID), lambda i: (0, 0)),
            pl.BlockSpec((din, D_HID), lambda i: (0, 0)),
            pl.BlockSpec((1, D_HID), lambda i: (0, 0)),
        ],
        out_specs=pl.BlockSpec((4, BM, 128), lambda i: (0, i, 0)),
        out_shape=jax.ShapeDtypeStruct((4, NP, 128), jnp.float32),
    )(x0_blk, x1_blk, x2_blk, wa, wb, wc, bias)


def _head_body(x_ref, wl_ref, bl_ref, o_ref):
    x = jnp.concatenate([x_ref[k] for k in range(4)], axis=1)
    logits = jnp.dot(x, wl_ref[...], preferred_element_type=jnp.float32)
    logits = logits + bl_ref[...]
    m = jnp.max(logits, axis=1, keepdims=True)
    z = logits - m
    lse = jnp.log(jnp.sum(jnp.exp(z), axis=1, keepdims=True))
    o_ref[...] = z - lse


def _head(x_blk, wlin, blin):
    return pl.pallas_call(
        _head_body,
        grid=(NP // BM,),
        in_specs=[
            pl.BlockSpec((4, BM, 128), lambda i: (0, i, 0)),
            pl.BlockSpec((D_HID, D_OUT), lambda i: (0, 0)),
            pl.BlockSpec((1, D_OUT), lambda i: (0, 0)),
        ],
        out_specs=pl.BlockSpec((BM, D_OUT), lambda i: (i, 0)),
        out_shape=jax.ShapeDtypeStruct((NP, D_OUT), jnp.float32),
    )(x_blk, wlin, blin)


def _to_blocked(x):
    npad, d = x.shape
    return x.reshape(npad, d // 128, 128).transpose(1, 0, 2)


def kernel(x, edge_index, W1_0, W1_1, W1_2, b1, W2_0, W2_1, W2_2, b2,
           Wlin, blin):
    row = edge_index[0]
    col = edge_index[1]

    norm, norm2 = _norm_kernel(row, col)

    x_pad = jnp.pad(x, ((0, NP - N), (0, 0)))
    x_blk = _to_blocked(x_pad)
    x_flat = x_blk.reshape(2 * NP, 128)

    tx1 = _prop2(x_flat, row, col, norm)
    p2 = _prop2(tx1, row, col, norm2)
    h1 = _mm3(x_blk, tx1.reshape(2, NP, 128), p2.reshape(2, NP, 128),
              W1_0, W1_1, W1_2, b1.reshape(1, D_HID))

    h1_flat = h1.reshape(4 * NP, 128)
    s1 = _prop4(h1_flat, row, col, norm)
    p22 = _prop4(s1, row, col, norm2)
    h2 = _mm3(h1, s1.reshape(4, NP, 128), p22.reshape(4, NP, 128),
              W2_0, W2_1, W2_2, b2.reshape(1, D_HID))

    out = _head(h2, Wlin, blin.reshape(1, D_OUT))
    return out[:N]


# speedup vs baseline: 8.1098x; 2.4701x over previous
import functools

import jax
import jax.numpy as jnp
from jax import lax
from jax.experimental import pallas as pl
from jax.experimental.pallas import tpu as pltpu
from jax.experimental.pallas import tpu_sc as plsc

N = 10000
E = 160000
D_IN = 256
D_HID = 512
D_OUT = 40

NP = 10240
L = 16
NC = 2
NS = 16
EPT = E // NS
CH = 80
NCHUNK = EPT // CH
RPT = NP // NS
WB = 64
NWB = RPT // WB

_mesh = plsc.VectorSubcoreMesh(core_axis_name="c", subcore_axis_name="s")
_sc_params = pltpu.CompilerParams(needs_layout_passes=False)


def _i32v(val):
    return lax.broadcast(jnp.int32(val), (L,))


def _f32v(val):
    return lax.broadcast(jnp.float32(val), (L,))


def _rsqrt16(d):
    xi = plsc.bitcast(d, jnp.int32)
    mi = _i32v(0x5F3759DF) - lax.shift_right_arithmetic(xi, _i32v(1))
    y = plsc.bitcast(mi, jnp.float32)
    for _ in range(3):
        y = y * (_f32v(1.5) - _f32v(0.5) * d * y * y)
    return y


def _norm_body(row2_hbm, col2_hbm, dinv_hbm, sl_hbm,
               deg_sh, sl_sh, rows_st, cols_st, zv, ones_v,
               ridx0, ridx1, vb0, vb1, dv640,
               sd0, sd1, sv0, sv1):
    c = lax.axis_index("c")
    s = lax.axis_index("s")
    ridx = [ridx0, ridx1]
    vb = [vb0, vb1]
    sd = [sd0, sd1]
    sv = [sv0, sv1]
    zero16 = _f32v(0.0)
    one16 = _f32v(1.0)

    def fill_z(i, carry):
        zv[pl.ds(i * L, L)] = zero16
        return carry
    lax.fori_loop(0, RPT // L, fill_z, 0)
    for i in range(CH // L):
        ones_v[pl.ds(i * L, L)] = one16

    pltpu.sync_copy(zv, deg_sh.at[pl.ds(s * RPT, RPT)])
    pltpu.sync_copy(zv, sl_sh.at[pl.ds(s * RPT, RPT)])

    pltpu.sync_copy(row2_hbm.at[s], rows_st)
    pltpu.sync_copy(col2_hbm.at[s], cols_st)
    plsc.subcore_barrier()

    def _deg_start(p):
        pltpu.make_async_copy(ones_v, deg_sh.at[ridx[p]], sd[p]).start(add=True)
        pltpu.make_async_copy(vb[p], sl_sh.at[ridx[p]], sv[p]).start(add=True)

    def _deg_wait(p):
        pltpu.make_async_copy(ones_v, deg_sh.at[ridx[p]], sd[p]).wait()
        pltpu.make_async_copy(vb[p], sl_sh.at[ridx[p]], sv[p]).wait()

    def chunk(k, p):
        @pl.when(k >= 2)
        def _():
            _deg_wait(p)
        for j in range(CH // L):
            r16 = rows_st[k, pl.ds(j * L, L)]
            c16 = cols_st[k, pl.ds(j * L, L)]
            ridx[p][pl.ds(j * L, L)] = r16
            vb[p][pl.ds(j * L, L)] = jnp.where(r16 == c16, one16, zero16)
        _deg_start(p)

    def loop2(kk, carry):
        chunk(2 * kk, 0)
        chunk(2 * kk + 1, 1)
        return carry
    lax.fori_loop(0, (NCHUNK - 1) // 2, loop2, 0)
    chunk(NCHUNK - 1, 0)
    _deg_wait(1)
    _deg_wait(0)
    plsc.subcore_barrier()

    @pl.when(c == 0)
    def _():
        pltpu.sync_copy(deg_sh.at[pl.ds(s * RPT, RPT)], dv640)

        def dl(i, carry):
            d = dv640[pl.ds(i * L, L)]
            y = _rsqrt16(d)
            dv640[pl.ds(i * L, L)] = jnp.where(d > _f32v(0.5), y, zero16)
            return carry
        lax.fori_loop(0, RPT // L, dl, 0)
        pltpu.sync_copy(dv640, dinv_hbm.at[pl.ds(s * RPT, RPT)])
        pltpu.sync_copy(sl_sh.at[pl.ds(s * RPT, RPT)],
                        sl_hbm.at[pl.ds(s * RPT, RPT)])


_norm_kernel = pl.kernel(
    _norm_body,
    out_type=(jax.ShapeDtypeStruct((NP,), jnp.float32),
              jax.ShapeDtypeStruct((NP,), jnp.float32)),
    mesh=_mesh,
    compiler_params=_sc_params,
    scratch_types=[
        pltpu.VMEM_SHARED((NP,), jnp.float32),
        pltpu.VMEM_SHARED((NP,), jnp.float32),
        pltpu.VMEM((NCHUNK, CH), jnp.int32),
        pltpu.VMEM((NCHUNK, CH), jnp.int32),
        pltpu.VMEM((RPT,), jnp.float32),
        pltpu.VMEM((CH,), jnp.float32),
        pltpu.VMEM((CH,), jnp.int32),
        pltpu.VMEM((CH,), jnp.int32),
        pltpu.VMEM((CH,), jnp.float32),
        pltpu.VMEM((CH,), jnp.float32),
        pltpu.VMEM((RPT,), jnp.float32),
        pltpu.SemaphoreType.DMA,
        pltpu.SemaphoreType.DMA,
        pltpu.SemaphoreType.DMA,
        pltpu.SemaphoreType.DMA,
    ],
)


def _prop_body(nb, dscale, emit_g, *args):
    n_out = 2 if emit_g else 1
    g_hbm, h_hbm, row_hbm, col_hbm, dinv_hbm, sl_hbm = args[:6]
    tx_hbm = args[6]
    gn_hbm = args[7] if emit_g else None
    scr = list(args[6 + n_out:])
    acc_sh = scr.pop(0)
    rr = [scr.pop(0) for _ in range(4)]
    cc = [scr.pop(0) for _ in range(4)]
    gx = [scr.pop(0) for _ in range(3)]
    dinv640 = scr.pop(0)
    sl640 = scr.pop(0)
    rb = [scr.pop(0) for _ in range(3)]
    og0 = scr.pop(0) if emit_g else None
    si = [scr.pop(0) for _ in range(4)]
    sg = [scr.pop(0) for _ in range(3)]
    ss = [scr.pop(0) for _ in range(3)]
    sot = scr.pop(0)
    sog = scr.pop(0) if emit_g else None

    c = lax.axis_index("c")
    s = lax.axis_index("s")
    zero16 = _f32v(0.0)
    ndsc = _f32v(-float(dscale))
    ebase = s * EPT
    KL = NCHUNK - 1

    pltpu.sync_copy(dinv_hbm.at[pl.ds(s * RPT, RPT)], dinv640)
    pltpu.sync_copy(sl_hbm.at[pl.ds(s * RPT, RPT)], sl640)

    def idx_start(k, u4):
        off = ebase + k * CH
        pltpu.make_async_copy(row_hbm.at[pl.ds(off, CH)], rr[u4], si[u4]).start()
        pltpu.make_async_copy(col_hbm.at[pl.ds(off, CH)], cc[u4], si[u4]).start()

    def idx_wait(k, u4):
        off = ebase + k * CH
        pltpu.make_async_copy(row_hbm.at[pl.ds(off, CH)], rr[u4], si[u4]).wait()
        pltpu.make_async_copy(col_hbm.at[pl.ds(off, CH)], cc[u4], si[u4]).wait()

    for bi in range(nb // NC):
        b = c + NC * bi
        bvec = lax.broadcast(b * NP, (L,))

        def zb(i, carry):
            for g in range(128 // L):
                rb[2][i, pl.ds(g * L, L)] = zero16
            return carry
        lax.fori_loop(0, CH, zb, 0)
        for t in range(RPT // CH):
            pltpu.sync_copy(rb[2], acc_sh.at[pl.ds(s * RPT + t * CH, CH), :])
        plsc.subcore_barrier()

        def gx_compute(u4, u3):
            for j in range(CH // L):
                gx[u3][pl.ds(j * L, L)] = rr[u4][pl.ds(j * L, L)] + bvec

        def g_start(u3):
            pltpu.make_async_copy(g_hbm.at[gx[u3]], rb[u3], sg[u3]).start()

        def g_wait(u3):
            pltpu.make_async_copy(g_hbm.at[gx[u3]], rb[u3], sg[u3]).wait()

        def s_start(u3, u4):
            pltpu.make_async_copy(
                rb[u3], acc_sh.at[cc[u4]], ss[u3]).start(add=True)

        def s_wait(u3, u4):
            pltpu.make_async_copy(rb[u3], acc_sh.at[cc[u4]], ss[u3]).wait()

        def chunk(k, u3, u4):
            g_wait(u3)
            s_start(u3, u4)
            u3n = (u3 + 2) % 3
            u4n2 = (u4 + 2) % 4
            u4n3 = (u4 + 3) % 4

            @pl.when(k + 2 <= KL)
            def _():
                idx_wait(k + 2, u4n2)
                gx_compute(u4n2, u3n)

                @pl.when(k >= 1)
                def _():
                    s_wait(u3n, u4n3)
                g_start(u3n)

            @pl.when(k + 3 <= KL)
            def _():
                idx_start(k + 3, u4n3)

        idx_start(0, 0)
        idx_start(1, 1)
        idx_start(2, 2)
        idx_wait(0, 0)
        gx_compute(0, 0)
        idx_wait(1, 1)
        gx_compute(1, 1)
        g_start(0)
        g_start(1)

        def loop12(kk, carry):
            k0 = kk * 12
            for d in range(12):
                chunk(k0 + d, d % 3, d % 4)
            return carry
        lax.fori_loop(0, 10, loop12, 0)
        chunk(120, 0, 0)
        chunk(121, 1, 1)
        chunk(122, 2, 2)
        chunk(123, 0, 3)
        chunk(124, 1, 0)
        s_wait(2, 2)
        s_wait(0, 3)
        s_wait(1, 0)
        plsc.subcore_barrier()

        def out_start(t):
            pltpu.make_async_copy(
                rb[2], tx_hbm.at[pl.ds(b * NP + s * RPT + t * CH, CH), :],
                sot).start()
            if emit_g:
                pltpu.make_async_copy(
                    og0, gn_hbm.at[pl.ds(b * NP + s * RPT + t * CH, CH), :],
                    sog).start()

        def out_wait(t):
            pltpu.make_async_copy(
                rb[2], tx_hbm.at[pl.ds(b * NP + s * RPT + t * CH, CH), :],
                sot).wait()
            if emit_g:
                pltpu.make_async_copy(
                    og0, gn_hbm.at[pl.ds(b * NP + s * RPT + t * CH, CH), :],
                    sog).wait()

        for t in range(RPT // CH):
            pltpu.sync_copy(acc_sh.at[pl.ds(s * RPT + t * CH, CH), :], rb[0])
            pltpu.sync_copy(
                h_hbm.at[pl.ds(b * NP + s * RPT + t * CH, CH), :], rb[1])
            if t >= 1:
                out_wait(t - 1)

            def wrow(v, carry):
                lv = t * CH + v
                nv = lax.broadcast(lv, (L,))
                dv = plsc.load_gather(dinv640, [nv])
                slv = plsc.load_gather(sl640, [nv])
                a = ndsc * dv
                bcoef = ndsc * slv
                for g in range(128 // L):
                    txv = (a * rb[0][v, pl.ds(g * L, L)]
                           + bcoef * rb[1][v, pl.ds(g * L, L)])
                    rb[2][v, pl.ds(g * L, L)] = txv
                    if emit_g:
                        og0[v, pl.ds(g * L, L)] = dv * txv
                return carry
            lax.fori_loop(0, CH, wrow, 0)
            out_start(t)
        out_wait(RPT // CH - 1)


def _make_prop(nb, dscale, emit_g):
    outs = [jax.ShapeDtypeStruct((nb * NP, 128), jnp.float32)]
    if emit_g:
        outs.append(jax.ShapeDtypeStruct((nb * NP, 128), jnp.float32))
    scratch = [pltpu.VMEM_SHARED((NP, 128), jnp.float32)]
    scratch += [pltpu.VMEM((CH,), jnp.int32)] * 4
    scratch += [pltpu.VMEM((CH,), jnp.int32)] * 4
    scratch += [pltpu.VMEM((CH,), jnp.int32)] * 3
    scratch += [pltpu.VMEM((RPT,), jnp.float32)] * 2
    scratch += [pltpu.VMEM((CH, 128), jnp.float32)] * 3
    if emit_g:
        scratch.append(pltpu.VMEM((CH, 128), jnp.float32))
    nsem = 4 + 3 + 3 + 1 + (1 if emit_g else 0)
    scratch += [pltpu.SemaphoreType.DMA] * nsem
    return pl.kernel(
        functools.partial(_prop_body, nb, dscale, emit_g),
        out_type=tuple(outs) if emit_g else outs[0],
        mesh=_mesh,
        compiler_params=_sc_params,
        scratch_types=scratch,
    )


_prop2_first = _make_prop(2, 1, True)
_prop2_second = _make_prop(2, 2, False)
_prop4_first = _make_prop(4, 1, True)
_prop4_second = _make_prop(4, 2, False)


BM = 512


def _elem0_body(x_ref, dinv_ref, o_ref):
    d = dinv_ref[...]
    for k in range(x_ref.shape[0]):
        o_ref[k] = d * x_ref[k]


def _elem0(x_blk, dinv_col):
    nb = x_blk.shape[0]
    return pl.pallas_call(
        _elem0_body,
        grid=(NP // BM,),
        in_specs=[
            pl.BlockSpec((nb, BM, 128), lambda i: (0, i, 0)),
            pl.BlockSpec((BM, 1), lambda i: (i, 0)),
        ],
        out_specs=pl.BlockSpec((nb, BM, 128), lambda i: (0, i, 0)),
        out_shape=jax.ShapeDtypeStruct((nb, NP, 128), jnp.float32),
    )(x_blk, dinv_col)


def _mm3_body(emit_g, x0_ref, x1_ref, x2_ref, wa_ref, wb_ref, wc_ref, b_ref,
              dinv_ref, *o_refs):
    o_ref = o_refs[0]
    nbin = x0_ref.shape[0]
    x0 = jnp.concatenate([x0_ref[k] for k in range(nbin)], axis=1)
    x1 = jnp.concatenate([x1_ref[k] for k in range(nbin)], axis=1)
    x2 = jnp.concatenate([x2_ref[k] for k in range(nbin)], axis=1)
    wc = wc_ref[...]
    bf = jnp.bfloat16
    acc = jnp.dot(x0.astype(bf), (wa_ref[...] - wc).astype(bf),
                  preferred_element_type=jnp.float32)
    acc = acc + jnp.dot(x1.astype(bf), wb_ref[...].astype(bf),
                        preferred_element_type=jnp.float32)
    acc = acc + jnp.dot(x2.astype(bf), wc.astype(bf),
                        preferred_element_type=jnp.float32)
    acc = acc + b_ref[...]
    h = jnp.maximum(acc, 0.0)
    for k in range(o_ref.shape[0]):
        o_ref[k] = h[:, k * 128:(k + 1) * 128]
    if emit_g:
        d = dinv_ref[...]
        for k in range(o_ref.shape[0]):
            o_refs[1][k] = d * h[:, k * 128:(k + 1) * 128]


def _mm3(emit_g, x0_blk, x1_blk, x2_blk, wa, wb, wc, bias, dinv_col):
    nbin = x0_blk.shape[0]
    din = nbin * 128
    out_shape = [jax.ShapeDtypeStruct((4, NP, 128), jnp.float32)]
    out_specs = [pl.BlockSpec((4, BM, 128), lambda i: (0, i, 0))]
    if emit_g:
        out_shape.append(jax.ShapeDtypeStruct((4, NP, 128), jnp.float32))
        out_specs.append(pl.BlockSpec((4, BM, 128), lambda i: (0, i, 0)))
    return pl.pallas_call(
        functools.partial(_mm3_body, emit_g),
        grid=(NP // BM,),
        in_specs=[
            pl.BlockSpec((nbin, BM, 128), lambda i: (0, i, 0)),
            pl.BlockSpec((nbin, BM, 128), lambda i: (0, i, 0)),
            pl.BlockSpec((nbin, BM, 128), lambda i: (0, i, 0)),
            pl.BlockSpec((din, D_HID), lambda i: (0, 0)),
            pl.BlockSpec((din, D_HID), lambda i: (0, 0)),
            pl.BlockSpec((din, D_HID), lambda i: (0, 0)),
            pl.BlockSpec((1, D_HID), lambda i: (0, 0)),
            pl.BlockSpec((BM, 1), lambda i: (i, 0)),
        ],
        out_specs=out_specs if emit_g else out_specs[0],
        out_shape=tuple(out_shape) if emit_g else out_shape[0],
    )(x0_blk, x1_blk, x2_blk, wa, wb, wc, bias, dinv_col)


def _mm3_head_body(x0_ref, x1_ref, x2_ref, wa_ref, wb_ref, wc_ref, b_ref,
                   wl_ref, bl_ref, o_ref):
    nbin = x0_ref.shape[0]
    x0 = jnp.concatenate([x0_ref[k] for k in range(nbin)], axis=1)
    x1 = jnp.concatenate([x1_ref[k] for k in range(nbin)], axis=1)
    x2 = jnp.concatenate([x2_ref[k] for k in range(nbin)], axis=1)
    wc = wc_ref[...]
    bf = jnp.bfloat16
    acc = jnp.dot(x0.astype(bf), (wa_ref[...] - wc).astype(bf),
                  preferred_element_type=jnp.float32)
    acc = acc + jnp.dot(x1.astype(bf), wb_ref[...].astype(bf),
                        preferred_element_type=jnp.float32)
    acc = acc + jnp.dot(x2.astype(bf), wc.astype(bf),
                        preferred_element_type=jnp.float32)
    acc = acc + b_ref[...]
    h = jnp.maximum(acc, 0.0)
    logits = jnp.dot(h.astype(jnp.bfloat16), wl_ref[...].astype(jnp.bfloat16),
                     preferred_element_type=jnp.float32)
    logits = logits + bl_ref[...]
    m = jnp.max(logits, axis=1, keepdims=True)
    z = logits - m
    lse = jnp.log(jnp.sum(jnp.exp(z), axis=1, keepdims=True))
    o_ref[...] = z - lse


def _mm3_head(x0_blk, x1_blk, x2_blk, wa, wb, wc, bias, wlin, blin):
    nbin = x0_blk.shape[0]
    din = nbin * 128
    return pl.pallas_call(
        _mm3_head_body,
        grid=(NP // BM,),
        in_specs=[
            pl.BlockSpec((nbin, BM, 128), lambda i: (0, i, 0)),
            pl.BlockSpec((nbin, BM, 128), lambda i: (0, i, 0)),
            pl.BlockSpec((nbin, BM, 128), lambda i: (0, i, 0)),
            pl.BlockSpec((din, D_HID), lambda i: (0, 0)),
            pl.BlockSpec((din, D_HID), lambda i: (0, 0)),
            pl.BlockSpec((din, D_HID), lambda i: (0, 0)),
            pl.BlockSpec((1, D_HID), lambda i: (0, 0)),
            pl.BlockSpec((D_HID, D_OUT), lambda i: (0, 0)),
            pl.BlockSpec((1, D_OUT), lambda i: (0, 0)),
        ],
        out_specs=pl.BlockSpec((BM, D_OUT), lambda i: (i, 0)),
        out_shape=jax.ShapeDtypeStruct((NP, D_OUT), jnp.float32),
    )(x0_blk, x1_blk, x2_blk, wa, wb, wc, bias, wlin, blin)


def kernel(x, edge_index, W1_0, W1_1, W1_2, b1, W2_0, W2_1, W2_2, b2,
           Wlin, blin):
    row = edge_index[0]
    col = edge_index[1]
    row2 = row.reshape(NS, NCHUNK, CH)
    col2 = col.reshape(NS, NCHUNK, CH)

    dinv, sl = _norm_kernel(row2, col2)
    dinv_col = dinv.reshape(NP, 1)

    x_pad = jnp.pad(x, ((0, NP - N), (0, 0)))
    x_blk = x_pad.reshape(NP, 2, 128).transpose(1, 0, 2)
    x_flat = x_blk.reshape(2 * NP, 128)

    g0 = _elem0(x_blk, dinv_col)
    tx1, g2 = _prop2_first(g0.reshape(2 * NP, 128), x_flat,
                           row, col, dinv, sl)
    tx2p = _prop2_second(g2, tx1, row, col, dinv, sl)
    h1, gh1 = _mm3(True, x_blk, tx1.reshape(2, NP, 128),
                   tx2p.reshape(2, NP, 128),
                   W1_0, W1_1, W1_2, b1.reshape(1, D_HID), dinv_col)

    tx1b, g2b = _prop4_first(gh1.reshape(4 * NP, 128), h1.reshape(4 * NP, 128),
                             row, col, dinv, sl)
    tx2pb = _prop4_second(g2b, tx1b, row, col, dinv, sl)
    out = _mm3_head(h1, tx1b.reshape(4, NP, 128), tx2pb.reshape(4, NP, 128),
                    W2_0, W2_1, W2_2, b2.reshape(1, D_HID),
                    Wlin, blin.reshape(1, D_OUT))
    return out[:N]


# speedup vs baseline: 8.2814x; 1.0212x over previous
import functools

import jax
import jax.numpy as jnp
from jax import lax
from jax.experimental import pallas as pl
from jax.experimental.pallas import tpu as pltpu
from jax.experimental.pallas import tpu_sc as plsc

N = 10000
E = 160000
D_IN = 256
D_HID = 512
D_OUT = 40

NP = 10240
L = 16
NC = 2
NS = 16
EPT = E // NS
CH = 80
NCHUNK = EPT // CH
RPT = NP // NS
WB = 64
NWB = RPT // WB

_mesh = plsc.VectorSubcoreMesh(core_axis_name="c", subcore_axis_name="s")
_sc_params = pltpu.CompilerParams(needs_layout_passes=False)


def _i32v(val):
    return lax.broadcast(jnp.int32(val), (L,))


def _f32v(val):
    return lax.broadcast(jnp.float32(val), (L,))


def _rsqrt16(d):
    xi = plsc.bitcast(d, jnp.int32)
    mi = _i32v(0x5F3759DF) - lax.shift_right_arithmetic(xi, _i32v(1))
    y = plsc.bitcast(mi, jnp.float32)
    for _ in range(3):
        y = y * (_f32v(1.5) - _f32v(0.5) * d * y * y)
    return y


def _norm_body(row2_hbm, col2_hbm, dinv_hbm, sl_hbm,
               deg_sh, sl_sh, rows_st, cols_st, zv, ones_v,
               ridx0, ridx1, vb0, vb1, dv640,
               sd0, sd1, sv0, sv1):
    c = lax.axis_index("c")
    s = lax.axis_index("s")
    ridx = [ridx0, ridx1]
    vb = [vb0, vb1]
    sd = [sd0, sd1]
    sv = [sv0, sv1]
    zero16 = _f32v(0.0)
    one16 = _f32v(1.0)

    def fill_z(i, carry):
        zv[pl.ds(i * L, L)] = zero16
        return carry
    lax.fori_loop(0, RPT // L, fill_z, 0)
    for i in range(CH // L):
        ones_v[pl.ds(i * L, L)] = one16

    pltpu.sync_copy(zv, deg_sh.at[pl.ds(s * RPT, RPT)])
    pltpu.sync_copy(zv, sl_sh.at[pl.ds(s * RPT, RPT)])

    pltpu.sync_copy(row2_hbm.at[s], rows_st)
    pltpu.sync_copy(col2_hbm.at[s], cols_st)
    plsc.subcore_barrier()

    def _deg_start(p):
        pltpu.make_async_copy(ones_v, deg_sh.at[ridx[p]], sd[p]).start(add=True)
        pltpu.make_async_copy(vb[p], sl_sh.at[ridx[p]], sv[p]).start(add=True)

    def _deg_wait(p):
        pltpu.make_async_copy(ones_v, deg_sh.at[ridx[p]], sd[p]).wait()
        pltpu.make_async_copy(vb[p], sl_sh.at[ridx[p]], sv[p]).wait()

    def chunk(k, p):
        @pl.when(k >= 2)
        def _():
            _deg_wait(p)
        for j in range(CH // L):
            r16 = rows_st[k, pl.ds(j * L, L)]
            c16 = cols_st[k, pl.ds(j * L, L)]
            ridx[p][pl.ds(j * L, L)] = r16
            vb[p][pl.ds(j * L, L)] = jnp.where(r16 == c16, one16, zero16)
        _deg_start(p)

    def loop2(kk, carry):
        chunk(2 * kk, 0)
        chunk(2 * kk + 1, 1)
        return carry
    lax.fori_loop(0, (NCHUNK - 1) // 2, loop2, 0)
    chunk(NCHUNK - 1, 0)
    _deg_wait(1)
    _deg_wait(0)
    plsc.subcore_barrier()

    @pl.when(c == 0)
    def _():
        pltpu.sync_copy(deg_sh.at[pl.ds(s * RPT, RPT)], dv640)

        def dl(i, carry):
            d = dv640[pl.ds(i * L, L)]
            y = _rsqrt16(d)
            dv640[pl.ds(i * L, L)] = jnp.where(d > _f32v(0.5), y, zero16)
            return carry
        lax.fori_loop(0, RPT // L, dl, 0)
        pltpu.sync_copy(dv640, dinv_hbm.at[pl.ds(s * RPT, RPT)])
        pltpu.sync_copy(sl_sh.at[pl.ds(s * RPT, RPT)],
                        sl_hbm.at[pl.ds(s * RPT, RPT)])


_norm_kernel = pl.kernel(
    _norm_body,
    out_type=(jax.ShapeDtypeStruct((NP,), jnp.float32),
              jax.ShapeDtypeStruct((NP,), jnp.float32)),
    mesh=_mesh,
    compiler_params=_sc_params,
    scratch_types=[
        pltpu.VMEM_SHARED((NP,), jnp.float32),
        pltpu.VMEM_SHARED((NP,), jnp.float32),
        pltpu.VMEM((NCHUNK, CH), jnp.int32),
        pltpu.VMEM((NCHUNK, CH), jnp.int32),
        pltpu.VMEM((RPT,), jnp.float32),
        pltpu.VMEM((CH,), jnp.float32),
        pltpu.VMEM((CH,), jnp.int32),
        pltpu.VMEM((CH,), jnp.int32),
        pltpu.VMEM((CH,), jnp.float32),
        pltpu.VMEM((CH,), jnp.float32),
        pltpu.VMEM((RPT,), jnp.float32),
        pltpu.SemaphoreType.DMA,
        pltpu.SemaphoreType.DMA,
        pltpu.SemaphoreType.DMA,
        pltpu.SemaphoreType.DMA,
    ],
)


def _prop_body(nb, dscale, emit_g, *args):
    n_out = 2 if emit_g else 1
    g_hbm, h_hbm, row_hbm, col_hbm, dinv_hbm, sl_hbm = args[:6]
    tx_hbm = args[6]
    gn_hbm = args[7] if emit_g else None
    scr = list(args[6 + n_out:])
    acc_sh = scr.pop(0)
    rr = [scr.pop(0) for _ in range(4)]
    cc = [scr.pop(0) for _ in range(4)]
    gx = [scr.pop(0) for _ in range(3)]
    dinv640 = scr.pop(0)
    sl640 = scr.pop(0)
    rb = [scr.pop(0) for _ in range(3)]
    og0 = scr.pop(0) if emit_g else None
    si = [scr.pop(0) for _ in range(4)]
    sg = [scr.pop(0) for _ in range(3)]
    ss = [scr.pop(0) for _ in range(3)]
    sot = scr.pop(0)
    shm = scr.pop(0)
    sog = scr.pop(0) if emit_g else None

    c = lax.axis_index("c")
    s = lax.axis_index("s")
    zero16 = _f32v(0.0)
    ndsc = _f32v(-float(dscale))
    ebase = s * EPT
    KL = NCHUNK - 1

    pltpu.sync_copy(dinv_hbm.at[pl.ds(s * RPT, RPT)], dinv640)
    pltpu.sync_copy(sl_hbm.at[pl.ds(s * RPT, RPT)], sl640)

    def idx_start(k, u4):
        off = ebase + k * CH
        pltpu.make_async_copy(row_hbm.at[pl.ds(off, CH)], rr[u4], si[u4]).start()
        pltpu.make_async_copy(col_hbm.at[pl.ds(off, CH)], cc[u4], si[u4]).start()

    def idx_wait(k, u4):
        off = ebase + k * CH
        pltpu.make_async_copy(row_hbm.at[pl.ds(off, CH)], rr[u4], si[u4]).wait()
        pltpu.make_async_copy(col_hbm.at[pl.ds(off, CH)], cc[u4], si[u4]).wait()

    for bi in range(nb // NC):
        b = c + NC * bi
        bvec = lax.broadcast(b * NP, (L,))

        def zb(i, carry):
            for g in range(128 // L):
                rb[2][i, pl.ds(g * L, L)] = zero16
            return carry
        lax.fori_loop(0, CH, zb, 0)
        for t in range(RPT // CH):
            pltpu.sync_copy(rb[2], acc_sh.at[pl.ds(s * RPT + t * CH, CH), :])
        plsc.subcore_barrier()

        def gx_compute(u4, u3):
            for j in range(CH // L):
                gx[u3][pl.ds(j * L, L)] = rr[u4][pl.ds(j * L, L)] + bvec

        def g_start(u3):
            pltpu.make_async_copy(g_hbm.at[gx[u3]], rb[u3], sg[u3]).start()

        def g_wait(u3):
            pltpu.make_async_copy(g_hbm.at[gx[u3]], rb[u3], sg[u3]).wait()

        def s_start(u3, u4):
            pltpu.make_async_copy(
                rb[u3], acc_sh.at[cc[u4]], ss[u3]).start(add=True)

        def s_wait(u3, u4):
            pltpu.make_async_copy(rb[u3], acc_sh.at[cc[u4]], ss[u3]).wait()

        def chunk(k, u3, u4):
            g_wait(u3)
            s_start(u3, u4)
            u3n = (u3 + 2) % 3
            u4n2 = (u4 + 2) % 4
            u4n3 = (u4 + 3) % 4

            @pl.when(k + 2 <= KL)
            def _():
                idx_wait(k + 2, u4n2)
                gx_compute(u4n2, u3n)

                @pl.when(k >= 1)
                def _():
                    s_wait(u3n, u4n3)
                g_start(u3n)

            @pl.when(k + 3 <= KL)
            def _():
                idx_start(k + 3, u4n3)

        idx_start(0, 0)
        idx_start(1, 1)
        idx_start(2, 2)
        idx_wait(0, 0)
        gx_compute(0, 0)
        idx_wait(1, 1)
        gx_compute(1, 1)
        g_start(0)
        g_start(1)

        def loop12(kk, carry):
            k0 = kk * 12
            for d in range(12):
                chunk(k0 + d, d % 3, d % 4)
            return carry
        lax.fori_loop(0, 10, loop12, 0)
        chunk(120, 0, 0)
        chunk(121, 1, 1)
        chunk(122, 2, 2)
        chunk(123, 0, 3)
        chunk(124, 1, 0)
        s_wait(2, 2)
        s_wait(0, 3)
        s_wait(1, 0)
        plsc.subcore_barrier()

        def out_start(t):
            pltpu.make_async_copy(
                rb[2], tx_hbm.at[pl.ds(b * NP + s * RPT + t * CH, CH), :],
                sot).start()
            if emit_g:
                pltpu.make_async_copy(
                    og0, gn_hbm.at[pl.ds(b * NP + s * RPT + t * CH, CH), :],
                    sog).start()

        def out_wait(t):
            pltpu.make_async_copy(
                rb[2], tx_hbm.at[pl.ds(b * NP + s * RPT + t * CH, CH), :],
                sot).wait()
            if emit_g:
                pltpu.make_async_copy(
                    og0, gn_hbm.at[pl.ds(b * NP + s * RPT + t * CH, CH), :],
                    sog).wait()

        def h_start(t):
            pltpu.make_async_copy(
                h_hbm.at[pl.ds(b * NP + s * RPT + t * CH, CH), :], rb[1],
                shm).start()

        def h_wait(t):
            pltpu.make_async_copy(
                h_hbm.at[pl.ds(b * NP + s * RPT + t * CH, CH), :], rb[1],
                shm).wait()

        h_start(0)
        for t in range(RPT // CH):
            pltpu.sync_copy(acc_sh.at[pl.ds(s * RPT + t * CH, CH), :], rb[0])
            h_wait(t)
            if t >= 1:
                out_wait(t - 1)

            def wrow(v, carry):
                lv = t * CH + v
                nv = lax.broadcast(lv, (L,))
                dv = plsc.load_gather(dinv640, [nv])
                slv = plsc.load_gather(sl640, [nv])
                a = ndsc * dv
                bcoef = ndsc * slv
                for g in range(128 // L):
                    txv = (a * rb[0][v, pl.ds(g * L, L)]
                           + bcoef * rb[1][v, pl.ds(g * L, L)])
                    rb[2][v, pl.ds(g * L, L)] = txv
                    if emit_g:
                        og0[v, pl.ds(g * L, L)] = dv * txv
                return carry
            lax.fori_loop(0, CH, wrow, 0)
            if t + 1 < RPT // CH:
                h_start(t + 1)
            out_start(t)
        out_wait(RPT // CH - 1)


def _make_prop(nb, dscale, emit_g):
    outs = [jax.ShapeDtypeStruct((nb * NP, 128), jnp.float32)]
    if emit_g:
        outs.append(jax.ShapeDtypeStruct((nb * NP, 128), jnp.float32))
    scratch = [pltpu.VMEM_SHARED((NP, 128), jnp.float32)]
    scratch += [pltpu.VMEM((CH,), jnp.int32)] * 4
    scratch += [pltpu.VMEM((CH,), jnp.int32)] * 4
    scratch += [pltpu.VMEM((CH,), jnp.int32)] * 3
    scratch += [pltpu.VMEM((RPT,), jnp.float32)] * 2
    scratch += [pltpu.VMEM((CH, 128), jnp.float32)] * 3
    if emit_g:
        scratch.append(pltpu.VMEM((CH, 128), jnp.float32))
    nsem = 4 + 3 + 3 + 2 + (1 if emit_g else 0)
    scratch += [pltpu.SemaphoreType.DMA] * nsem
    return pl.kernel(
        functools.partial(_prop_body, nb, dscale, emit_g),
        out_type=tuple(outs) if emit_g else outs[0],
        mesh=_mesh,
        compiler_params=_sc_params,
        scratch_types=scratch,
    )


_prop2_first = _make_prop(2, 1, True)
_prop2_second = _make_prop(2, 2, False)
_prop4_first = _make_prop(4, 1, True)
_prop4_second = _make_prop(4, 2, False)


BM = 512


def _elem0_body(x_ref, dinv_ref, o_ref):
    d = dinv_ref[...]
    for k in range(x_ref.shape[0]):
        o_ref[k] = d * x_ref[k]


def _elem0(x_blk, dinv_col):
    nb = x_blk.shape[0]
    return pl.pallas_call(
        _elem0_body,
        grid=(NP // BM,),
        in_specs=[
            pl.BlockSpec((nb, BM, 128), lambda i: (0, i, 0)),
            pl.BlockSpec((BM, 1), lambda i: (i, 0)),
        ],
        out_specs=pl.BlockSpec((nb, BM, 128), lambda i: (0, i, 0)),
        out_shape=jax.ShapeDtypeStruct((nb, NP, 128), jnp.float32),
    )(x_blk, dinv_col)


def _mm3_body(emit_g, x0_ref, x1_ref, x2_ref, wa_ref, wb_ref, wc_ref, b_ref,
              dinv_ref, *o_refs):
    o_ref = o_refs[0]
    nbin = x0_ref.shape[0]
    x0 = jnp.concatenate([x0_ref[k] for k in range(nbin)], axis=1)
    x1 = jnp.concatenate([x1_ref[k] for k in range(nbin)], axis=1)
    x2 = jnp.concatenate([x2_ref[k] for k in range(nbin)], axis=1)
    wc = wc_ref[...]
    bf = jnp.bfloat16
    acc = jnp.dot(x0.astype(bf), (wa_ref[...] - wc).astype(bf),
                  preferred_element_type=jnp.float32)
    acc = acc + jnp.dot(x1.astype(bf), wb_ref[...].astype(bf),
                        preferred_element_type=jnp.float32)
    acc = acc + jnp.dot(x2.astype(bf), wc.astype(bf),
                        preferred_element_type=jnp.float32)
    acc = acc + b_ref[...]
    h = jnp.maximum(acc, 0.0)
    for k in range(o_ref.shape[0]):
        o_ref[k] = h[:, k * 128:(k + 1) * 128]
    if emit_g:
        d = dinv_ref[...]
        for k in range(o_ref.shape[0]):
            o_refs[1][k] = d * h[:, k * 128:(k + 1) * 128]


def _mm3(emit_g, x0_blk, x1_blk, x2_blk, wa, wb, wc, bias, dinv_col):
    nbin = x0_blk.shape[0]
    din = nbin * 128
    out_shape = [jax.ShapeDtypeStruct((4, NP, 128), jnp.float32)]
    out_specs = [pl.BlockSpec((4, BM, 128), lambda i: (0, i, 0))]
    if emit_g:
        out_shape.append(jax.ShapeDtypeStruct((4, NP, 128), jnp.float32))
        out_specs.append(pl.BlockSpec((4, BM, 128), lambda i: (0, i, 0)))
    return pl.pallas_call(
        functools.partial(_mm3_body, emit_g),
        grid=(NP // BM,),
        in_specs=[
            pl.BlockSpec((nbin, BM, 128), lambda i: (0, i, 0)),
            pl.BlockSpec((nbin, BM, 128), lambda i: (0, i, 0)),
            pl.BlockSpec((nbin, BM, 128), lambda i: (0, i, 0)),
            pl.BlockSpec((din, D_HID), lambda i: (0, 0)),
            pl.BlockSpec((din, D_HID), lambda i: (0, 0)),
            pl.BlockSpec((din, D_HID), lambda i: (0, 0)),
            pl.BlockSpec((1, D_HID), lambda i: (0, 0)),
            pl.BlockSpec((BM, 1), lambda i: (i, 0)),
        ],
        out_specs=out_specs if emit_g else out_specs[0],
        out_shape=tuple(out_shape) if emit_g else out_shape[0],
    )(x0_blk, x1_blk, x2_blk, wa, wb, wc, bias, dinv_col)


def _mm3_head_body(x0_ref, x1_ref, x2_ref, wa_ref, wb_ref, wc_ref, b_ref,
                   wl_ref, bl_ref, o_ref):
    nbin = x0_ref.shape[0]
    x0 = jnp.concatenate([x0_ref[k] for k in range(nbin)], axis=1)
    x1 = jnp.concatenate([x1_ref[k] for k in range(nbin)], axis=1)
    x2 = jnp.concatenate([x2_ref[k] for k in range(nbin)], axis=1)
    wc = wc_ref[...]
    bf = jnp.bfloat16
    acc = jnp.dot(x0.astype(bf), (wa_ref[...] - wc).astype(bf),
                  preferred_element_type=jnp.float32)
    acc = acc + jnp.dot(x1.astype(bf), wb_ref[...].astype(bf),
                        preferred_element_type=jnp.float32)
    acc = acc + jnp.dot(x2.astype(bf), wc.astype(bf),
                        preferred_element_type=jnp.float32)
    acc = acc + b_ref[...]
    h = jnp.maximum(acc, 0.0)
    logits = jnp.dot(h.astype(jnp.bfloat16), wl_ref[...].astype(jnp.bfloat16),
                     preferred_element_type=jnp.float32)
    logits = logits + bl_ref[...]
    m = jnp.max(logits, axis=1, keepdims=True)
    z = logits - m
    lse = jnp.log(jnp.sum(jnp.exp(z), axis=1, keepdims=True))
    o_ref[...] = z - lse


def _mm3_head(x0_blk, x1_blk, x2_blk, wa, wb, wc, bias, wlin, blin):
    nbin = x0_blk.shape[0]
    din = nbin * 128
    return pl.pallas_call(
        _mm3_head_body,
        grid=(NP // BM,),
        in_specs=[
            pl.BlockSpec((nbin, BM, 128), lambda i: (0, i, 0)),
            pl.BlockSpec((nbin, BM, 128), lambda i: (0, i, 0)),
            pl.BlockSpec((nbin, BM, 128), lambda i: (0, i, 0)),
            pl.BlockSpec((din, D_HID), lambda i: (0, 0)),
            pl.BlockSpec((din, D_HID), lambda i: (0, 0)),
            pl.BlockSpec((din, D_HID), lambda i: (0, 0)),
            pl.BlockSpec((1, D_HID), lambda i: (0, 0)),
            pl.BlockSpec((D_HID, D_OUT), lambda i: (0, 0)),
            pl.BlockSpec((1, D_OUT), lambda i: (0, 0)),
        ],
        out_specs=pl.BlockSpec((BM, D_OUT), lambda i: (i, 0)),
        out_shape=jax.ShapeDtypeStruct((NP, D_OUT), jnp.float32),
    )(x0_blk, x1_blk, x2_blk, wa, wb, wc, bias, wlin, blin)


def kernel(x, edge_index, W1_0, W1_1, W1_2, b1, W2_0, W2_1, W2_2, b2,
           Wlin, blin):
    row = edge_index[0]
    col = edge_index[1]
    row2 = row.reshape(NS, NCHUNK, CH)
    col2 = col.reshape(NS, NCHUNK, CH)

    dinv, sl = _norm_kernel(row2, col2)
    dinv_col = dinv.reshape(NP, 1)

    x_pad = jnp.pad(x, ((0, NP - N), (0, 0)))
    x_blk = x_pad.reshape(NP, 2, 128).transpose(1, 0, 2)
    x_flat = x_blk.reshape(2 * NP, 128)

    g0 = _elem0(x_blk, dinv_col)
    tx1, g2 = _prop2_first(g0.reshape(2 * NP, 128), x_flat,
                           row, col, dinv, sl)
    tx2p = _prop2_second(g2, tx1, row, col, dinv, sl)
    h1, gh1 = _mm3(True, x_blk, tx1.reshape(2, NP, 128),
                   tx2p.reshape(2, NP, 128),
                   W1_0, W1_1, W1_2, b1.reshape(1, D_HID), dinv_col)

    tx1b, g2b = _prop4_first(gh1.reshape(4 * NP, 128), h1.reshape(4 * NP, 128),
                             row, col, dinv, sl)
    tx2pb = _prop4_second(g2b, tx1b, row, col, dinv, sl)
    out = _mm3_head(h1, tx1b.reshape(4, NP, 128), tx2pb.reshape(4, NP, 128),
                    W2_0, W2_1, W2_2, b2.reshape(1, D_HID),
                    Wlin, blin.reshape(1, D_OUT))
    return out[:N]
